# baseline (speedup 1.0000x reference)
import functools

import jax
import jax.numpy as jnp
import numpy as np
from jax import lax
from jax.experimental import pallas as pl
from jax.experimental.pallas import tpu as pltpu
from jax.experimental.pallas import tpu_sc as plsc

N = 10000
E = 320000
D = 128

NC = 2
NS = 16
NW = NC * NS
G = 128

CH0 = 24
CH1 = 136
TCH = NS * (CH0 + CH1)
TCH_ARR = TCH + 8
E_PAD = TCH_ARR * G
N_PAD = 10240
RPS = N_PAD // NS


def _sc_agg_build(want_deg):
    mesh = plsc.VectorSubcoreMesh(
        core_axis_name="c", subcore_axis_name="s", num_cores=NC, num_subcores=NS
    )
    out_type = [jax.ShapeDtypeStruct((NC, N_PAD, D), jnp.float32)]
    scratch = [
        pltpu.VMEM((48, G), jnp.int32),
        pltpu.VMEM((48, G), jnp.int32),
        pltpu.VMEM((2, G, D), jnp.float32),
        pltpu.VMEM_SHARED((N_PAD, D), jnp.float32),
        [pltpu.SemaphoreType.DMA] * 2,
    ]
    if want_deg:
        out_type.append(jax.ShapeDtypeStruct((NC, N_PAD), jnp.float32))
        scratch += [
            pltpu.VMEM((G,), jnp.float32),
            pltpu.VMEM_SHARED((N_PAD,), jnp.float32),
        ]

    def body(y_hbm, src_hbm, dst_hbm, *refs):
        if want_deg:
            agg_out, deg_out, src_v, dst_v, rows_v, agg_sh, sems, ones_v, deg_sh = refs
        else:
            agg_out, src_v, dst_v, rows_v, agg_sh, sems = refs
        c = lax.axis_index("c")
        s = lax.axis_index("s")

        def zrow(i, carry):
            for k in range(D // 16):
                rows_v[0, i, pl.ds(k * 16, 16)] = jnp.zeros((16,), jnp.float32)
            return carry

        lax.fori_loop(0, G, zrow, 0)
        for r in range(RPS // G):
            pltpu.sync_copy(rows_v.at[0], agg_sh.at[pl.ds(s * RPS + r * G, G)])
        if want_deg:
            for r in range(RPS // D):
                pltpu.sync_copy(
                    rows_v.at[0, 0], deg_sh.at[pl.ds(s * RPS + r * D, D)]
                )
            for k in range(G // 16):
                ones_v[pl.ds(k * 16, 16)] = jnp.ones((16,), jnp.float32)
        plsc.subcore_barrier()

        H0, O0 = (8, 8, 8), (0, 8, 16)
        H1, O1 = (48, 48, 40), (0, 48, 96)
        for h in range(3):
            base_h = jnp.where(
                c == 0, s * CH0 + O0[h], NS * CH0 + s * CH1 + O1[h]
            )
            pltpu.sync_copy(src_hbm.at[pl.ds(base_h, 48)], src_v)
            pltpu.sync_copy(dst_hbm.at[pl.ds(base_h, 48)], dst_v)

            @pl.when(c == 0)
            def _():
                Q = H0[h] // 2
                for t in range(2):
                    pltpu.async_copy(
                        y_hbm.at[src_v.at[t * Q]], rows_v.at[t], sems[t]
                    )

                def rchunk(jj, carry):
                    for t in range(2):
                        j = t * Q + jj
                        pltpu.make_async_copy(
                            y_hbm.at[pl.ds(0, G)], rows_v.at[t], sems[t]
                        ).wait()
                        pltpu.sync_copy(
                            rows_v.at[t], agg_sh.at[dst_v.at[j]], add=True
                        )
                        if want_deg:
                            pltpu.sync_copy(
                                ones_v, deg_sh.at[dst_v.at[j]], add=True
                            )

                        @pl.when(jj + 1 < Q)
                        def _():
                            pltpu.async_copy(
                                y_hbm.at[src_v.at[j + 1]], rows_v.at[t], sems[t]
                            )

                    return carry

                lax.fori_loop(0, Q, rchunk, 0)

            @pl.when(c == 1)
            def _():
                def schunk(j, carry):
                    pltpu.async_copy(
                        y_hbm.at[src_v.at[j]], rows_v.at[0], sems[0]
                    ).wait()
                    pltpu.sync_copy(rows_v.at[0], agg_sh.at[dst_v.at[j]], add=True)
                    if want_deg:
                        pltpu.sync_copy(ones_v, deg_sh.at[dst_v.at[j]], add=True)
                    return carry

                lax.fori_loop(0, H1[h], schunk, 0)

        plsc.subcore_barrier()

        pltpu.sync_copy(
            agg_sh.at[pl.ds(s * RPS, RPS)], agg_out.at[c, pl.ds(s * RPS, RPS)]
        )
        if want_deg:
            pltpu.sync_copy(
                deg_sh.at[pl.ds(s * RPS, RPS)], deg_out.at[c, pl.ds(s * RPS, RPS)]
            )

    return pl.kernel(body, out_type=out_type, mesh=mesh, scratch_types=scratch)


_sc_agg_deg = _sc_agg_build(True)
_sc_agg = _sc_agg_build(False)


BN = 2000
_GRID = N // BN


def _mm_body(x_ref, w_ref, o_ref):
    o_ref[...] = lax.dot_general(
        x_ref[...], w_ref[...], (((1,), (1,)), ((), ())),
        preferred_element_type=jnp.float32,
    )


_mm = pl.pallas_call(
    _mm_body,
    grid=(_GRID,),
    in_specs=[
        pl.BlockSpec((BN, D), lambda i: (i, 0)),
        pl.BlockSpec((D, D), lambda i: (0, 0)),
    ],
    out_specs=pl.BlockSpec((BN, D), lambda i: (i, 0)),
    out_shape=jax.ShapeDtypeStruct((N, D), jnp.float32),
)


def _combine_mm_body(p_ref, y_ref, dg_ref, b_ref, w_ref, o_ref):
    agg = p_ref[0] + p_ref[1]
    deg = dg_ref[0] + dg_ref[1] + 1.0
    h = (agg + y_ref[...]) / deg + b_ref[...]
    h = jnp.where(h >= 0.0, h, 0.01 * h)
    o_ref[...] = lax.dot_general(
        h, w_ref[...], (((1,), (1,)), ((), ())),
        preferred_element_type=jnp.float32,
    )


_combine_mm = pl.pallas_call(
    _combine_mm_body,
    grid=(_GRID,),
    in_specs=[
        pl.BlockSpec((NC, BN, D), lambda i: (0, i, 0)),
        pl.BlockSpec((BN, D), lambda i: (i, 0)),
        pl.BlockSpec((NC, BN, 1), lambda i: (0, i, 0)),
        pl.BlockSpec((1, D), lambda i: (0, 0)),
        pl.BlockSpec((D, D), lambda i: (0, 0)),
    ],
    out_specs=pl.BlockSpec((BN, D), lambda i: (i, 0)),
    out_shape=jax.ShapeDtypeStruct((N, D), jnp.float32),
)


def _combine_body(p_ref, y_ref, dg_ref, b_ref, o_ref):
    agg = p_ref[0] + p_ref[1]
    deg = dg_ref[0] + dg_ref[1] + 1.0
    h = (agg + y_ref[...]) / deg + b_ref[...]
    o_ref[...] = jnp.where(h >= 0.0, h, 0.01 * h)


_combine = pl.pallas_call(
    _combine_body,
    grid=(_GRID,),
    in_specs=[
        pl.BlockSpec((NC, BN, D), lambda i: (0, i, 0)),
        pl.BlockSpec((BN, D), lambda i: (i, 0)),
        pl.BlockSpec((NC, BN, 1), lambda i: (0, i, 0)),
        pl.BlockSpec((1, D), lambda i: (0, 0)),
    ],
    out_specs=pl.BlockSpec((BN, D), lambda i: (i, 0)),
    out_shape=jax.ShapeDtypeStruct((N, D), jnp.float32),
)


def kernel(feat, edge_index, W1, b1, W2, b2):
    ei = jnp.asarray(edge_index, jnp.int32)
    pad = E_PAD - E
    di = np.arange(pad, dtype=np.int32)
    src = jnp.concatenate([ei[0], jnp.asarray(di % N)]).reshape(TCH_ARR, G)
    dst = jnp.concatenate([ei[1], jnp.asarray(N + di % (N_PAD - N))]).reshape(TCH_ARR, G)
    b1r = b1.reshape(1, D)
    b2r = b2.reshape(1, D)

    y1 = _mm(feat, W1)
    p1, dg = _sc_agg_deg(y1, src, dst)
    dg3 = dg.reshape(NC, N_PAD, 1)
    y2 = _combine_mm(p1, y1, dg3, b1r, W2)
    p2 = _sc_agg(y2, src, dst)
    if isinstance(p2, (tuple, list)):
        p2 = p2[0]
    out = _combine(p2, y2, dg3, b2r)
    return out


# speedup vs baseline: 2.2727x; 2.2727x over previous
import functools

import jax
import jax.numpy as jnp
import numpy as np
from jax import lax
from jax.experimental import pallas as pl
from jax.experimental.pallas import tpu as pltpu
from jax.experimental.pallas import tpu_sc as plsc

N = 10000
E = 320000
D = 128

NC = 2
NS = 16
NW = NC * NS
G = 128

CH0 = 80
CH1 = 80
TCH = NS * (CH0 + CH1)
TCH_ARR = TCH + 16
E_PAD = TCH_ARR * G
N_PAD = 10240
RPS = N_PAD // NS


def _sc_agg_build(want_deg):
    mesh = plsc.VectorSubcoreMesh(
        core_axis_name="c", subcore_axis_name="s", num_cores=NC, num_subcores=NS
    )
    out_type = [jax.ShapeDtypeStruct((NC, N_PAD, D), jnp.float32)]
    scratch = [
        pltpu.VMEM((48, G), jnp.int32),
        pltpu.VMEM((48, G), jnp.int32),
        pltpu.VMEM((2, G, D), jnp.float32),
        pltpu.VMEM_SHARED((N_PAD, D), jnp.float32),
        [pltpu.SemaphoreType.DMA] * 2,
    ]
    if want_deg:
        out_type.append(jax.ShapeDtypeStruct((NC, N_PAD), jnp.float32))
        scratch += [
            pltpu.VMEM((G,), jnp.float32),
            pltpu.VMEM_SHARED((N_PAD,), jnp.float32),
        ]

    def body(y_hbm, src_hbm, dst_hbm, *refs):
        if want_deg:
            agg_out, deg_out, src_v, dst_v, rows_v, agg_sh, sems, ones_v, deg_sh = refs
        else:
            agg_out, src_v, dst_v, rows_v, agg_sh, sems = refs
        c = lax.axis_index("c")
        s = lax.axis_index("s")

        def zrow(i, carry):
            for k in range(D // 16):
                rows_v[0, i, pl.ds(k * 16, 16)] = jnp.zeros((16,), jnp.float32)
            return carry

        lax.fori_loop(0, G, zrow, 0)
        for r in range(RPS // G):
            pltpu.sync_copy(rows_v.at[0], agg_sh.at[pl.ds(s * RPS + r * G, G)])
        if want_deg:
            for r in range(RPS // D):
                pltpu.sync_copy(
                    rows_v.at[0, 0], deg_sh.at[pl.ds(s * RPS + r * D, D)]
                )
            for k in range(G // 16):
                ones_v[pl.ds(k * 16, 16)] = jnp.ones((16,), jnp.float32)
        plsc.subcore_barrier()

        H, O = (48, 32), (0, 48)
        for h in range(2):
            base_h = jnp.where(
                c == 0, s * CH0 + O[h], NS * CH0 + s * CH1 + O[h]
            )
            pltpu.sync_copy(src_hbm.at[pl.ds(base_h, 48)], src_v)
            pltpu.sync_copy(dst_hbm.at[pl.ds(base_h, 48)], dst_v)

            Q = H[h] // 2
            for t in range(2):
                pltpu.async_copy(
                    y_hbm.at[src_v.at[t * Q]], rows_v.at[t], sems[t]
                )

            def rchunk(jj, carry):
                for t in range(2):
                    j = t * Q + jj
                    pltpu.make_async_copy(
                        y_hbm.at[pl.ds(0, G)], rows_v.at[t], sems[t]
                    ).wait()
                    pltpu.sync_copy(
                        rows_v.at[t], agg_sh.at[dst_v.at[j]], add=True
                    )
                    if want_deg:
                        pltpu.sync_copy(ones_v, deg_sh.at[dst_v.at[j]], add=True)

                    @pl.when(jj + 1 < Q)
                    def _():
                        pltpu.async_copy(
                            y_hbm.at[src_v.at[j + 1]], rows_v.at[t], sems[t]
                        )

                return carry

            lax.fori_loop(0, Q, rchunk, 0)

        plsc.subcore_barrier()

        pltpu.sync_copy(
            agg_sh.at[pl.ds(s * RPS, RPS)], agg_out.at[c, pl.ds(s * RPS, RPS)]
        )
        if want_deg:
            pltpu.sync_copy(
                deg_sh.at[pl.ds(s * RPS, RPS)], deg_out.at[c, pl.ds(s * RPS, RPS)]
            )

    return pl.kernel(body, out_type=out_type, mesh=mesh, scratch_types=scratch)


_sc_agg_deg = _sc_agg_build(True)
_sc_agg = _sc_agg_build(False)


BN = 2000
_GRID = N // BN


def _mm_body(x_ref, w_ref, o_ref):
    o_ref[...] = lax.dot_general(
        x_ref[...], w_ref[...], (((1,), (1,)), ((), ())),
        preferred_element_type=jnp.float32,
    )


_mm = pl.pallas_call(
    _mm_body,
    grid=(_GRID,),
    in_specs=[
        pl.BlockSpec((BN, D), lambda i: (i, 0)),
        pl.BlockSpec((D, D), lambda i: (0, 0)),
    ],
    out_specs=pl.BlockSpec((BN, D), lambda i: (i, 0)),
    out_shape=jax.ShapeDtypeStruct((N, D), jnp.float32),
)


def _combine_mm_body(p_ref, y_ref, dg_ref, b_ref, w_ref, o_ref):
    agg = p_ref[0] + p_ref[1]
    deg = dg_ref[0] + dg_ref[1] + 1.0
    h = (agg + y_ref[...]) / deg + b_ref[...]
    h = jnp.where(h >= 0.0, h, 0.01 * h)
    o_ref[...] = lax.dot_general(
        h, w_ref[...], (((1,), (1,)), ((), ())),
        preferred_element_type=jnp.float32,
    )


_combine_mm = pl.pallas_call(
    _combine_mm_body,
    grid=(_GRID,),
    in_specs=[
        pl.BlockSpec((NC, BN, D), lambda i: (0, i, 0)),
        pl.BlockSpec((BN, D), lambda i: (i, 0)),
        pl.BlockSpec((NC, BN, 1), lambda i: (0, i, 0)),
        pl.BlockSpec((1, D), lambda i: (0, 0)),
        pl.BlockSpec((D, D), lambda i: (0, 0)),
    ],
    out_specs=pl.BlockSpec((BN, D), lambda i: (i, 0)),
    out_shape=jax.ShapeDtypeStruct((N, D), jnp.float32),
)


def _combine_body(p_ref, y_ref, dg_ref, b_ref, o_ref):
    agg = p_ref[0] + p_ref[1]
    deg = dg_ref[0] + dg_ref[1] + 1.0
    h = (agg + y_ref[...]) / deg + b_ref[...]
    o_ref[...] = jnp.where(h >= 0.0, h, 0.01 * h)


_combine = pl.pallas_call(
    _combine_body,
    grid=(_GRID,),
    in_specs=[
        pl.BlockSpec((NC, BN, D), lambda i: (0, i, 0)),
        pl.BlockSpec((BN, D), lambda i: (i, 0)),
        pl.BlockSpec((NC, BN, 1), lambda i: (0, i, 0)),
        pl.BlockSpec((1, D), lambda i: (0, 0)),
    ],
    out_specs=pl.BlockSpec((BN, D), lambda i: (i, 0)),
    out_shape=jax.ShapeDtypeStruct((N, D), jnp.float32),
)


def kernel(feat, edge_index, W1, b1, W2, b2):
    ei = jnp.asarray(edge_index, jnp.int32)
    pad = E_PAD - E
    di = np.arange(pad, dtype=np.int32)
    src = jnp.concatenate([ei[0], jnp.asarray(di % N)]).reshape(TCH_ARR, G)
    dst = jnp.concatenate([ei[1], jnp.asarray(N + di % (N_PAD - N))]).reshape(TCH_ARR, G)
    b1r = b1.reshape(1, D)
    b2r = b2.reshape(1, D)

    y1 = _mm(feat, W1)
    p1, dg = _sc_agg_deg(y1, src, dst)
    dg3 = dg.reshape(NC, N_PAD, 1)
    y2 = _combine_mm(p1, y1, dg3, b1r, W2)
    p2 = _sc_agg(y2, src, dst)
    if isinstance(p2, (tuple, list)):
        p2 = p2[0]
    out = _combine(p2, y2, dg3, b2r)
    return out
